# baseline (speedup 1.0000x reference)
import functools

import jax
import jax.numpy as jnp
from jax import lax
from jax.experimental import pallas as pl
from jax.experimental.pallas import tpu as pltpu
from jax.experimental.pallas import tpu_sc as plsc

NUM_CORES = 2
NUM_SUBCORES = 16
NW = NUM_CORES * NUM_SUBCORES

EMBED_DIM = 256
BATCH = 4096
SEQ = 50
SLABS_PER_W = BATCH // NW
SEQ_PAD = 64
NBUF = 4


def _sc_gather(x3d, table):
    mesh = plsc.VectorSubcoreMesh(core_axis_name="c", subcore_axis_name="s")

    @functools.partial(
        pl.kernel,
        mesh=mesh,
        out_type=jax.ShapeDtypeStruct((BATCH, SEQ, EMBED_DIM), jnp.float32),
        scratch_types=[
            pltpu.VMEM((SLABS_PER_W, SEQ_PAD), jnp.int32),
            pltpu.VMEM((SEQ_PAD, EMBED_DIM), jnp.float32),
            pltpu.VMEM((SEQ_PAD, EMBED_DIM), jnp.float32),
            pltpu.VMEM((SEQ_PAD, EMBED_DIM), jnp.float32),
            pltpu.VMEM((SEQ_PAD, EMBED_DIM), jnp.float32),
            pltpu.SemaphoreType.DMA,
            pltpu.SemaphoreType.DMA,
            pltpu.SemaphoreType.DMA,
            pltpu.SemaphoreType.DMA,
            pltpu.SemaphoreType.DMA,
            pltpu.SemaphoreType.DMA,
            pltpu.SemaphoreType.DMA,
            pltpu.SemaphoreType.DMA,
        ],
    )
    def k(x_hbm, table_hbm, out_hbm, idx_v, r0, r1, r2, r3,
          gs0, gs1, gs2, gs3, os0, os1, os2, os3):
        wid = lax.axis_index("s") * NUM_CORES + lax.axis_index("c")
        slab_base = wid * SLABS_PER_W
        bufs = (r0, r1, r2, r3)
        gsems = (gs0, gs1, gs2, gs3)
        osems = (os0, os1, os2, os3)

        pltpu.sync_copy(x_hbm.at[wid], idx_v)

        SEQ_LO = 48

        def gather_start(j, b):
            pltpu.async_copy(table_hbm.at[idx_v.at[j]], bufs[b], gsems[b])

        def gather_wait(b):
            pltpu.make_async_copy(
                table_hbm.at[idx_v.at[0]], bufs[b], gsems[b]).wait()

        def write_start(j, b):
            pltpu.async_copy(
                bufs[b].at[pl.ds(0, SEQ_LO)],
                out_hbm.at[slab_base + j, pl.ds(0, SEQ_LO)], osems[b])
            pltpu.async_copy(
                bufs[b].at[pl.ds(SEQ_LO, SEQ - SEQ_LO)],
                out_hbm.at[slab_base + j, pl.ds(SEQ_LO, SEQ - SEQ_LO)],
                osems[b])

        def write_wait(b):
            pltpu.make_async_copy(
                bufs[b].at[pl.ds(0, SEQ_LO)],
                out_hbm.at[slab_base, pl.ds(0, SEQ_LO)], osems[b]).wait()
            pltpu.make_async_copy(
                bufs[b].at[pl.ds(SEQ_LO, SEQ - SEQ_LO)],
                out_hbm.at[slab_base, pl.ds(SEQ_LO, SEQ - SEQ_LO)],
                osems[b]).wait()

        gather_start(0, 0)
        gather_start(1, 1)
        gather_start(2, 2)
        gather_wait(0)
        write_start(0, 0)
        gather_start(3, 3)

        def body(go, carry):
            for bb in range(NBUF):
                j = go * NBUF + bb + 1
                b = (bb + 1) % NBUF
                gather_wait(b)
                write_start(j, b)
                write_wait((b + 3) % NBUF)
                gather_start(j + 3, (b + 3) % NBUF)
            return carry

        lax.fori_loop(0, (SLABS_PER_W - NBUF) // NBUF, body, 0)

        for j in range(SLABS_PER_W - 3, SLABS_PER_W):
            b = j % NBUF
            gather_wait(b)
            write_start(j, b)
        for b in range(NBUF):
            write_wait(b)

    return k(x3d, table)


def kernel(x, table):
    n, s = x.shape
    x3d = x.reshape(NW, SLABS_PER_W, SEQ).astype(jnp.int32)
    x3d = jnp.pad(x3d, ((0, 0), (0, 0), (0, SEQ_PAD - SEQ)))
    return _sc_gather(x3d, table)


# speedup vs baseline: 8.4854x; 8.4854x over previous
import functools

import jax
import jax.numpy as jnp
from jax import lax
from jax.experimental import pallas as pl
from jax.experimental.pallas import tpu as pltpu
from jax.experimental.pallas import tpu_sc as plsc

NUM_CORES = 2
NUM_SUBCORES = 16
NW = NUM_CORES * NUM_SUBCORES

EMBED_DIM = 256
BATCH = 4096
SEQ = 50
SLABS_PER_W = BATCH // NW
SEQ_PAD = 64
NBUF = 4


def _sc_gather(x3d, table):
    mesh = plsc.VectorSubcoreMesh(core_axis_name="c", subcore_axis_name="s")

    @functools.partial(
        pl.kernel,
        mesh=mesh,
        out_type=jax.ShapeDtypeStruct((BATCH, SEQ, EMBED_DIM), jnp.float32),
        scratch_types=[
            pltpu.VMEM((SLABS_PER_W, SEQ_PAD), jnp.int32),
            pltpu.VMEM((SEQ_PAD, EMBED_DIM), jnp.float32),
            pltpu.VMEM((SEQ_PAD, EMBED_DIM), jnp.float32),
            pltpu.VMEM((SEQ_PAD, EMBED_DIM), jnp.float32),
            pltpu.VMEM((SEQ_PAD, EMBED_DIM), jnp.float32),
            pltpu.SemaphoreType.DMA,
            pltpu.SemaphoreType.DMA,
            pltpu.SemaphoreType.DMA,
            pltpu.SemaphoreType.DMA,
            pltpu.SemaphoreType.DMA,
            pltpu.SemaphoreType.DMA,
            pltpu.SemaphoreType.DMA,
            pltpu.SemaphoreType.DMA,
        ],
    )
    def k(x_hbm, table_hbm, out_hbm, idx_v, r0, r1, r2, r3,
          gs0, gs1, gs2, gs3, os0, os1, os2, os3):
        wid = lax.axis_index("s") * NUM_CORES + lax.axis_index("c")
        slab_base = wid * SLABS_PER_W
        bufs = (r0, r1, r2, r3)
        gsems = (gs0, gs1, gs2, gs3)
        osems = (os0, os1, os2, os3)

        pltpu.sync_copy(x_hbm.at[wid], idx_v)

        SEQ_LO = 48

        def gather_start(j, b):
            pltpu.async_copy(table_hbm.at[idx_v.at[j]], bufs[b], gsems[b])

        def gather_wait(b):
            pltpu.make_async_copy(
                table_hbm.at[idx_v.at[0]], bufs[b], gsems[b]).wait()

        def write_start(j, b):
            pltpu.async_copy(
                bufs[b].at[pl.ds(0, SEQ_LO)],
                out_hbm.at[slab_base + j, pl.ds(0, SEQ_LO)], osems[b])
            pltpu.async_copy(
                bufs[b].at[pl.ds(SEQ_LO, SEQ - SEQ_LO)],
                out_hbm.at[slab_base + j, pl.ds(SEQ_LO, SEQ - SEQ_LO)],
                osems[b])

        def write_wait(b):
            pltpu.make_async_copy(
                bufs[b].at[pl.ds(0, SEQ_LO)],
                out_hbm.at[slab_base, pl.ds(0, SEQ_LO)], osems[b]).wait()
            pltpu.make_async_copy(
                bufs[b].at[pl.ds(SEQ_LO, SEQ - SEQ_LO)],
                out_hbm.at[slab_base, pl.ds(SEQ_LO, SEQ - SEQ_LO)],
                osems[b]).wait()

        gather_start(0, 0)
        gather_start(1, 1)
        gather_start(2, 2)
        gather_wait(0)
        write_start(0, 0)
        gather_start(3, 3)

        def body(go, carry):
            for bb in range(NBUF):
                j = go * NBUF + bb + 1
                b = (bb + 1) % NBUF
                gather_wait(b)
                write_start(j, b)
                write_wait((b + 3) % NBUF)
                gather_start(j + 3, (b + 3) % NBUF)
            return carry

        lax.fori_loop(0, (SLABS_PER_W - NBUF) // NBUF, body, 0)

        for j in range(SLABS_PER_W - 3, SLABS_PER_W):
            b = j % NBUF
            gather_wait(b)
            write_start(j, b)
        for b in range(NBUF):
            write_wait(b)

    return k(x3d, table)


def kernel(x, table):
    n, s = x.shape
    x3d = x.reshape(NW, SLABS_PER_W, SEQ).astype(jnp.int32)
    x3d = jnp.concatenate([x3d, x3d[:, :, : SEQ_PAD - SEQ]], axis=-1)
    return _sc_gather(x3d, table)


# speedup vs baseline: 8.5227x; 1.0044x over previous
import functools

import jax
import jax.numpy as jnp
from jax import lax
from jax.experimental import pallas as pl
from jax.experimental.pallas import tpu as pltpu
from jax.experimental.pallas import tpu_sc as plsc

NUM_CORES = 2
NUM_SUBCORES = 16
NW = NUM_CORES * NUM_SUBCORES

EMBED_DIM = 256
BATCH = 4096
SEQ = 50
SLABS_PER_W = BATCH // NW
SEQ_PAD = 64
SEQ_LO = 48
NBUF = 6


def _sc_gather(x3d, table):
    mesh = plsc.VectorSubcoreMesh(core_axis_name="c", subcore_axis_name="s")

    @functools.partial(
        pl.kernel,
        mesh=mesh,
        out_type=jax.ShapeDtypeStruct((BATCH, SEQ, EMBED_DIM), jnp.float32),
        scratch_types=(
            [pltpu.VMEM((SLABS_PER_W, SEQ_PAD), jnp.int32)]
            + [pltpu.VMEM((SEQ_PAD, EMBED_DIM), jnp.float32)] * NBUF
            + [pltpu.SemaphoreType.DMA] * (2 * NBUF)
        ),
    )
    def k(x_hbm, table_hbm, out_hbm, idx_v, *bufs_sems):
        bufs = bufs_sems[:NBUF]
        gsems = bufs_sems[NBUF:2 * NBUF]
        osems = bufs_sems[2 * NBUF:]
        wid = lax.axis_index("s") * NUM_CORES + lax.axis_index("c")
        slab_base = wid * SLABS_PER_W

        pltpu.sync_copy(x_hbm.at[wid], idx_v)

        def gather_start(j, b):
            pltpu.async_copy(table_hbm.at[idx_v.at[j]], bufs[b], gsems[b])

        def gather_wait(b):
            pltpu.make_async_copy(
                table_hbm.at[idx_v.at[0]], bufs[b], gsems[b]).wait()

        def write_start(j, b):
            pltpu.async_copy(
                bufs[b].at[pl.ds(0, SEQ_LO)],
                out_hbm.at[slab_base + j, pl.ds(0, SEQ_LO)], osems[b])
            pltpu.async_copy(
                bufs[b].at[pl.ds(SEQ_LO, SEQ - SEQ_LO)],
                out_hbm.at[slab_base + j, pl.ds(SEQ_LO, SEQ - SEQ_LO)],
                osems[b])

        def write_wait(b):
            pltpu.make_async_copy(
                bufs[b].at[pl.ds(0, SEQ_LO)],
                out_hbm.at[slab_base, pl.ds(0, SEQ_LO)], osems[b]).wait()
            pltpu.make_async_copy(
                bufs[b].at[pl.ds(SEQ_LO, SEQ - SEQ_LO)],
                out_hbm.at[slab_base, pl.ds(SEQ_LO, SEQ - SEQ_LO)],
                osems[b]).wait()

        def steady(j, b):
            gather_wait(b)
            write_start(j, b)
            write_wait((b + NBUF - 1) % NBUF)
            gather_start(j + NBUF - 1, (b + NBUF - 1) % NBUF)

        for b in range(NBUF - 1):
            gather_start(b, b)
        gather_wait(0)
        write_start(0, 0)
        gather_start(NBUF - 1, NBUF - 1)

        def body(go, carry):
            for bb in range(NBUF):
                j = go * NBUF + bb + 1
                steady(j, (bb + 1) % NBUF)
            return carry

        n_steady = SLABS_PER_W - NBUF
        n_loop = (n_steady // NBUF) * NBUF
        lax.fori_loop(0, n_steady // NBUF, body, 0)
        for j in range(n_loop + 1, n_steady + 1):
            steady(j, j % NBUF)

        for j in range(SLABS_PER_W - NBUF + 1, SLABS_PER_W):
            b = j % NBUF
            gather_wait(b)
            write_start(j, b)
        for b in range(NBUF):
            write_wait(b)

    return k(x3d, table)


def kernel(x, table):
    n, s = x.shape
    x3d = x.reshape(NW, SLABS_PER_W, SEQ).astype(jnp.int32)
    x3d = jnp.concatenate([x3d, x3d[:, :, : SEQ_PAD - SEQ]], axis=-1)
    return _sc_gather(x3d, table)
